# Initial kernel scaffold; baseline (speedup 1.0000x reference)
import jax
import jax.numpy as jnp
from jax.experimental import pallas as pl


def kernel(x, pos, params, batch, num_graphs):
    raise NotImplementedError("write your pallas kernel here")


# speedup vs baseline: 7.5961x; 7.5961x over previous
import functools

import jax
import jax.numpy as jnp
import numpy as np
from jax.experimental import pallas as pl
from jax.experimental.pallas import tpu as pltpu

N_G = 256
SRC = 16
NSHELL = 5
SHELL_F = SRC + 3
FLAT_F = NSHELL * SHELL_F
EH_F = 4 * SRC + FLAT_F
_CUT = 5.0
_MAX = 40.0
_BOUNDS = [(5.0, 10.0), (10.0, 20.0), (20.0, 40.0), (40.0, 80.0), (80.0, None)]
_RBF_C = np.linspace(_CUT, _MAX, 8).astype(np.float32)
_RBF_STEP = float(_RBF_C[1] - _RBF_C[0])
_RBF_G = float(1.0 / max(_RBF_STEP ** 2, 1e-6))


def _silu(x):
    return x * jax.nn.sigmoid(x)


def _ln(x, g, b, eps=1e-5):
    m = x.mean(-1, keepdims=True)
    v = ((x - m) ** 2).mean(-1, keepdims=True)
    return (x - m) / jnp.sqrt(v + eps) * g + b


def _onehots(batch_ref, n):
    bids = batch_ref[0, 0, :]
    iota_g = jax.lax.broadcasted_iota(jnp.int32, (n, N_G), 1)
    oh = (bids[:, None] == iota_g).astype(jnp.float32)
    iota_gt = jax.lax.broadcasted_iota(jnp.int32, (N_G, n), 0)
    oht = (iota_gt == bids[None, :]).astype(jnp.float32)
    return oh, oht


def _mm(a, b):
    return jnp.dot(a, b, preferred_element_type=jnp.float32)


def _k1_body(x_ref, pos_ref, batch_ref, g_ref, b_ref, w1_ref, b1_ref,
             w2_ref, b2_ref, src_out_ref, sums_ref):
    i = pl.program_id(0)
    xb = x_ref[...]
    h = _ln(xb, g_ref[...], b_ref[...])
    a = _silu(_mm(h, w1_ref[...]) + b1_ref[...])
    s = _mm(a, w2_ref[...]) + b2_ref[...]
    src_out_ref[...] = s
    _, oht = _onehots(batch_ref, xb.shape[0])
    ones = jnp.ones((xb.shape[0], 1), jnp.float32)
    vals = jnp.concatenate([s, pos_ref[...], ones], axis=1)

    @pl.when(i == 0)
    def _():
        sums_ref[...] = jnp.zeros_like(sums_ref)

    sums_ref[...] += _mm(oht, vals)


def _k2_body(src_ref, pos_ref, batch_ref, sums0_ref, sg_ref, sb_ref,
             kscr_ref, kw1_ref, kb1_ref, kw2_ref, kb2_ref,
             shell_ref, gmean_s, center_s):
    i = pl.program_id(0)

    @pl.when(i == 0)
    def _():
        s0 = sums0_ref[...]
        cc = jnp.clip(s0[:, 19:20], 1.0, None)
        gmean_s[...] = s0[:, 0:16] / cc
        center_s[...] = s0[:, 16:19] / cc
        shell_ref[...] = jnp.zeros_like(shell_ref)

    n = src_ref.shape[0]
    oh, oht = _onehots(batch_ref, n)
    srcn = _ln(src_ref[...] - _mm(oh, gmean_s[...]), sg_ref[...], sb_ref[...])
    d = pos_ref[...] - _mm(oh, center_s[...])
    r = jnp.sqrt(jnp.sum(d * d, axis=1))
    screening = jax.nn.softplus(kscr_ref[0, 0])
    base = jnp.exp(-screening * r) / jnp.maximum(r, 1e-6)
    rc = r[:, None]
    centers = _CUT + _RBF_STEP * jax.lax.broadcasted_iota(
        jnp.int32, (1, 8), 1).astype(jnp.float32)
    rbf = jnp.exp(-_RBF_G * (rc - centers) ** 2)
    gate_in = jnp.concatenate([rc / _CUT, rc / _MAX, rbf], axis=1)
    g1 = _silu(_mm(gate_in, kw1_ref[...]) + kb1_ref[...])
    lg = 1.0 + jnp.tanh((_mm(g1, kw2_ref[...]) + kb2_ref[...])[:, 0])
    w = base * lg

    masks = [((rc >= lo) if hi is None else
              ((rc >= lo) & (rc < hi))).astype(jnp.float32)
             for lo, hi in _BOUNDS]
    m_all = jnp.concatenate(masks, axis=1)
    wm = w[:, None] * m_all
    cols = [srcn * wm[:, s:s + 1] for s in range(NSHELL)]
    cols += [m_all, rc * m_all, (rc * rc) * m_all]
    vals = jnp.concatenate(cols, axis=1)
    shell_ref[...] += _mm(oht, vals)


def _k3_body(src_ref, batch_ref, sums0_ref, shell_ref, sg_ref, sb_ref,
             seg_ref, seb_ref, sw1_ref, sb1_ref, sw2_ref, sb2_ref,
             eg_ref, eb_ref, ew1_ref, eb1_ref, ew2_ref, eb2_ref,
             fg_ref, es_ref, out_ref, gmean_s, flat_s, ctx_s):
    i = pl.program_id(0)

    @pl.when(i == 0)
    def _():
        s0 = sums0_ref[...]
        cc = jnp.clip(s0[:, 19:20], 1.0, None)
        gmean_s[...] = s0[:, 0:16] / cc
        ss = shell_ref[...]
        cols = []
        for s in range(NSHELL):
            cnt = ss[:, 80 + s:81 + s]
            ccs = jnp.clip(cnt, 1.0, None)
            cols.append(ss[:, 16 * s:16 * s + 16] / ccs)
            cols.append(cnt)
            cols.append(ss[:, 85 + s:86 + s] / ccs)
            cols.append(jnp.sqrt(ss[:, 90 + s:91 + s] / ccs))
        flat = jnp.concatenate(cols, axis=1)
        flat_s[...] = flat
        t = _ln(flat, seg_ref[...], seb_ref[...])
        ctx = _mm(_silu(_mm(t, sw1_ref[...]) + sb1_ref[...]), sw2_ref[...])
        ctx_s[...] = ctx + sb2_ref[...]
        out_ref[...] = jnp.zeros_like(out_ref)

    n = src_ref.shape[0]
    oh, oht = _onehots(batch_ref, n)
    srcn = _ln(src_ref[...] - _mm(oh, gmean_s[...]), sg_ref[...], sb_ref[...])
    ctx_b = _mm(oh, ctx_s[...])
    flat_b = _mm(oh, flat_s[...])
    ein = jnp.concatenate(
        [srcn, ctx_b, srcn * ctx_b, srcn - ctx_b, flat_b], axis=1)
    t = _ln(ein, eg_ref[...], eb_ref[...])
    e = _mm(_silu(_mm(t, ew1_ref[...]) + eb1_ref[...]), ew2_ref[...])
    e = (e + eb2_ref[...]) * (fg_ref[0, 0] * jnp.exp(es_ref[0, 0]))
    out_ref[...] += _mm(oht, e)


def _row(v):
    return v.reshape(1, -1)


def _scalar(v):
    return jnp.asarray(v, jnp.float32).reshape(1, 1)


def kernel(x, pos, params, batch, num_graphs):
    p = params
    n = x.shape[0]
    blk = 2000
    for cand in (2000, 1000, 500, 200, 8):
        if n % cand == 0:
            blk = cand
            break
    nb = n // blk
    batch3 = batch.reshape(nb, 1, blk)

    def fixed(shape):
        return pl.BlockSpec(shape, lambda i: tuple(0 for _ in shape))

    node2 = lambda c: pl.BlockSpec((blk, c), lambda i: (i, 0))
    bspec = pl.BlockSpec((1, 1, blk), lambda i: (i, 0, 0))

    src_raw, sums0 = pl.pallas_call(
        _k1_body,
        grid=(nb,),
        in_specs=[node2(128), node2(3), bspec,
                  fixed((1, 128)), fixed((1, 128)),
                  fixed((128, 64)), fixed((1, 64)),
                  fixed((64, SRC)), fixed((1, SRC))],
        out_specs=[node2(SRC), fixed((N_G, 20))],
        out_shape=[jax.ShapeDtypeStruct((n, SRC), jnp.float32),
                   jax.ShapeDtypeStruct((N_G, 20), jnp.float32)],
    )(x, pos, batch3,
      _row(p['in_ln_g']), _row(p['in_ln_b']),
      p['src_W1'], _row(p['src_b1']),
      p['src_W2'], _row(p['src_b2']))

    shellsums = pl.pallas_call(
        _k2_body,
        grid=(nb,),
        in_specs=[node2(SRC), node2(3), bspec, fixed((N_G, 20)),
                  fixed((1, SRC)), fixed((1, SRC)), fixed((1, 1)),
                  fixed((10, 32)), fixed((1, 32)),
                  fixed((32, 1)), fixed((1, 1))],
        out_specs=fixed((N_G, FLAT_F)),
        out_shape=jax.ShapeDtypeStruct((N_G, FLAT_F), jnp.float32),
        scratch_shapes=[pltpu.VMEM((N_G, SRC), jnp.float32),
                        pltpu.VMEM((N_G, 3), jnp.float32)],
    )(src_raw, pos, batch3, sums0,
      _row(p['srcn_g']), _row(p['srcn_b']), _scalar(p['kscr']),
      p['kg_W1'], _row(p['kg_b1']), p['kg_W2'], _scalar(p['kg_b2']))

    out = pl.pallas_call(
        _k3_body,
        grid=(nb,),
        in_specs=[node2(SRC), bspec, fixed((N_G, 20)), fixed((N_G, FLAT_F)),
                  fixed((1, SRC)), fixed((1, SRC)),
                  fixed((1, FLAT_F)), fixed((1, FLAT_F)),
                  fixed((FLAT_F, 64)), fixed((1, 64)),
                  fixed((64, SRC)), fixed((1, SRC)),
                  fixed((1, EH_F)), fixed((1, EH_F)),
                  fixed((EH_F, 64)), fixed((1, 64)),
                  fixed((64, 1)), fixed((1, 1)),
                  fixed((1, 1)), fixed((1, 1))],
        out_specs=fixed((N_G, 1)),
        out_shape=jax.ShapeDtypeStruct((N_G, 1), jnp.float32),
        scratch_shapes=[pltpu.VMEM((N_G, SRC), jnp.float32),
                        pltpu.VMEM((N_G, FLAT_F), jnp.float32),
                        pltpu.VMEM((N_G, SRC), jnp.float32)],
    )(src_raw, batch3, sums0, shellsums,
      _row(p['srcn_g']), _row(p['srcn_b']),
      _row(p['se_ln_g']), _row(p['se_ln_b']),
      p['se_W1'], _row(p['se_b1']), p['se_W2'], _row(p['se_b2']),
      _row(p['eh_ln_g']), _row(p['eh_ln_b']),
      p['eh_W1'], _row(p['eh_b1']), p['eh_W2'], _scalar(p['eh_b2']),
      _scalar(p['far_gate']), _scalar(p['energy_scale']))

    return out[:, 0]


# speedup vs baseline: 21.7263x; 2.8602x over previous
import jax
import jax.numpy as jnp
import numpy as np
from jax.experimental import pallas as pl
from jax.experimental.pallas import tpu as pltpu

N_G = 256
SRC = 16
NSHELL = 5
FLAT_F = NSHELL * (SRC + 3)
EH_F = 4 * SRC + FLAT_F
_CUT = 5.0
_MAX = 40.0
_RBF_STEP = float(np.linspace(_CUT, _MAX, 8)[1] - np.linspace(_CUT, _MAX, 8)[0])
_RBF_G = float(1.0 / max(_RBF_STEP ** 2, 1e-6))


def _silu(x):
    return x * jax.nn.sigmoid(x)


def _ln_rows(x, g, b, eps=1e-5):
    m = x.mean(-1, keepdims=True)
    v = ((x - m) ** 2).mean(-1, keepdims=True)
    return (x - m) / jnp.sqrt(v + eps) * g + b


def _ln_cols(x, g, b, eps=1e-5):
    m = x.mean(0, keepdims=True)
    v = ((x - m) ** 2).mean(0, keepdims=True)
    return (x - m) / jnp.sqrt(v + eps) * g + b


def _oh_bmajor(batch_ref, n):
    bids = batch_ref[0, 0, :]
    iota = jax.lax.broadcasted_iota(jnp.int32, (n, N_G), 1)
    return (bids[:, None] == iota).astype(jnp.bfloat16)


def _oh_gmajor(batch_ref, n):
    bids = batch_ref[0]
    iota = jax.lax.broadcasted_iota(jnp.int32, (N_G, n), 0)
    return (iota == bids).astype(jnp.bfloat16)


def _mm(a, b):
    return jnp.dot(a, b, preferred_element_type=jnp.float32)


def _mm_nt(a, b):
    return jax.lax.dot_general(
        a, b, (((1,), (1,)), ((), ())), preferred_element_type=jnp.float32)


def _bf(x):
    return x.astype(jnp.bfloat16)


def _k1_body(x_ref, post_ref, batch_ref, g_ref, b_ref, w1_ref, b1_ref,
             w2t_ref, b2_ref, srct_out_ref, sums_ref):
    i = pl.program_id(0)
    xb = x_ref[...]
    h = _ln_rows(xb, g_ref[...], b_ref[...])
    a = _silu(_mm(_bf(h), _bf(w1_ref[...])) + b1_ref[...])
    src_t = _mm_nt(_bf(w2t_ref[...]), _bf(a)) + b2_ref[...]
    srct_out_ref[...] = src_t[None]
    oh = _oh_bmajor(batch_ref, xb.shape[0])
    ones = jnp.ones((1, xb.shape[0]), jnp.float32)
    vals = jnp.concatenate([src_t, post_ref[0], ones], axis=0)

    @pl.when(i == 0)
    def _():
        sums_ref[...] = jnp.zeros_like(sums_ref)

    sums_ref[...] += _mm(_bf(vals), oh)


def _k2_body(srct_ref, post_ref, batch_ref, sums0_ref, sg_ref, sb_ref,
             kscr_ref, kw1t_ref, kb1_ref, kw2t_ref, kb2_ref,
             shell_ref, srcn_out_ref, gmean_s, center_s):
    i = pl.program_id(0)

    @pl.when(i == 0)
    def _():
        s0 = sums0_ref[...]
        cc = jnp.clip(s0[19:20, :], 1.0, None)
        gmean_s[...] = s0[0:16, :] / cc
        center_s[...] = s0[16:19, :] / cc
        shell_ref[...] = jnp.zeros_like(shell_ref)

    n = srct_ref.shape[2]
    oht = _oh_gmajor(batch_ref, n)
    gmean_b = _mm(_bf(gmean_s[...]), oht)
    center_b = _mm(_bf(center_s[...]), oht)
    srcn = _ln_cols(srct_ref[0] - gmean_b, sg_ref[...], sb_ref[...])
    srcn_out_ref[...] = srcn[None]
    d = post_ref[0] - center_b
    r = jnp.sqrt(jnp.sum(d * d, axis=0, keepdims=True))
    screening = jax.nn.softplus(kscr_ref[0, 0])
    base = jnp.exp(-screening * r) / jnp.maximum(r, 1e-6)
    centers = _CUT + _RBF_STEP * jax.lax.broadcasted_iota(
        jnp.int32, (8, 1), 0).astype(jnp.float32)
    rbf = jnp.exp(-_RBF_G * (r - centers) ** 2)
    gate_in = jnp.concatenate([r / _CUT, r / _MAX, rbf], axis=0)
    g1 = _silu(_mm(kw1t_ref[...], gate_in) + kb1_ref[...])
    lg = 1.0 + jnp.tanh(_mm(kw2t_ref[...], g1) + kb2_ref[...])
    w = base * lg

    k5 = jax.lax.broadcasted_iota(jnp.int32, (NSHELL, 1), 0)
    lows = _CUT * jnp.exp2(k5.astype(jnp.float32))
    highs = jnp.where(k5 == NSHELL - 1, jnp.inf, 2.0 * lows)
    m_all = ((r >= lows) & (r < highs)).astype(jnp.float32)
    wm = w * m_all
    rows = [srcn * wm[s:s + 1, :] for s in range(NSHELL)]
    rows += [m_all, r * m_all, (r * r) * m_all]
    vals = jnp.concatenate(rows, axis=0)
    shell_ref[...] += _mm_nt(_bf(vals), oht)


def _k3_body(srcn_ref, batch_ref, shell_ref, seg_ref, seb_ref,
             sw1t_ref, sb1_ref, sw2t_ref, sb2_ref,
             eg_ref, eb_ref, ew1t_ref, eb1_ref, ew2t_ref, eb2_ref,
             fg_ref, es_ref, out_ref, flat_s, ctx_s):
    i = pl.program_id(0)

    @pl.when(i == 0)
    def _():
        ss = shell_ref[...]
        rows = []
        for s in range(NSHELL):
            cnt = ss[80 + s:81 + s, :]
            ccs = jnp.clip(cnt, 1.0, None)
            rows.append(ss[16 * s:16 * s + 16, :] / ccs)
            rows.append(cnt)
            rows.append(ss[85 + s:86 + s, :] / ccs)
            rows.append(jnp.sqrt(ss[90 + s:91 + s, :] / ccs))
        flat = jnp.concatenate(rows, axis=0)
        flat_s[...] = flat
        t = _ln_cols(flat, seg_ref[...], seb_ref[...])
        a = _silu(_mm(_bf(sw1t_ref[...]), _bf(t)) + sb1_ref[...])
        ctx_s[...] = _mm(_bf(sw2t_ref[...]), _bf(a)) + sb2_ref[...]
        out_ref[...] = jnp.zeros_like(out_ref)

    n = srcn_ref.shape[2]
    oht = _oh_gmajor(batch_ref, n)
    srcn = srcn_ref[0]
    ctx_b = _mm(_bf(ctx_s[...]), oht)
    flat_b = _mm(_bf(flat_s[...]), oht)
    ein = jnp.concatenate(
        [srcn, ctx_b, srcn * ctx_b, srcn - ctx_b, flat_b], axis=0)
    t = _ln_cols(ein, eg_ref[...], eb_ref[...])
    a = _silu(_mm(_bf(ew1t_ref[...]), _bf(t)) + eb1_ref[...])
    e = _mm(ew2t_ref[...], a) + eb2_ref[...]
    e = e * (fg_ref[0, 0] * jnp.exp(es_ref[0, 0]))
    out_ref[...] += _mm_nt(e, oht.astype(jnp.float32))


def _row(v):
    return v.reshape(1, -1)


def _col(v):
    return v.reshape(-1, 1)


def _scalar(v):
    return jnp.asarray(v, jnp.float32).reshape(1, 1)


def kernel(x, pos, params, batch, num_graphs):
    p = params
    n = x.shape[0]
    blk = 2000
    for cand in (2000, 1000, 500, 200, 8):
        if n % cand == 0:
            blk = cand
            break
    nb = n // blk
    batch3 = batch.reshape(nb, 1, blk)
    pos_t3 = pos.reshape(nb, blk, 3).transpose(0, 2, 1)

    def fixed(shape):
        return pl.BlockSpec(shape, lambda i: tuple(0 for _ in shape))

    xspec = pl.BlockSpec((blk, 128), lambda i: (i, 0))
    node3 = lambda c: pl.BlockSpec((1, c, blk), lambda i: (i, 0, 0))

    src_t3, sums0 = pl.pallas_call(
        _k1_body,
        grid=(nb,),
        in_specs=[xspec, node3(3), node3(1),
                  fixed((1, 128)), fixed((1, 128)),
                  fixed((128, 64)), fixed((1, 64)),
                  fixed((SRC, 64)), fixed((SRC, 1))],
        out_specs=[node3(SRC), fixed((20, N_G))],
        out_shape=[jax.ShapeDtypeStruct((nb, SRC, blk), jnp.float32),
                   jax.ShapeDtypeStruct((20, N_G), jnp.float32)],
    )(x, pos_t3, batch3,
      _row(p['in_ln_g']), _row(p['in_ln_b']),
      p['src_W1'], _row(p['src_b1']),
      p['src_W2'].T, _col(p['src_b2']))

    shellsums, srcn_t3 = pl.pallas_call(
        _k2_body,
        grid=(nb,),
        in_specs=[node3(SRC), node3(3), node3(1), fixed((20, N_G)),
                  fixed((SRC, 1)), fixed((SRC, 1)), fixed((1, 1)),
                  fixed((32, 10)), fixed((32, 1)),
                  fixed((1, 32)), fixed((1, 1))],
        out_specs=[fixed((FLAT_F, N_G)), node3(SRC)],
        out_shape=[jax.ShapeDtypeStruct((FLAT_F, N_G), jnp.float32),
                   jax.ShapeDtypeStruct((nb, SRC, blk), jnp.float32)],
        scratch_shapes=[pltpu.VMEM((SRC, N_G), jnp.float32),
                        pltpu.VMEM((3, N_G), jnp.float32)],
    )(src_t3, pos_t3, batch3, sums0,
      _col(p['srcn_g']), _col(p['srcn_b']), _scalar(p['kscr']),
      p['kg_W1'].T, _col(p['kg_b1']), p['kg_W2'].T, _scalar(p['kg_b2']))

    out = pl.pallas_call(
        _k3_body,
        grid=(nb,),
        in_specs=[node3(SRC), node3(1), fixed((FLAT_F, N_G)),
                  fixed((FLAT_F, 1)), fixed((FLAT_F, 1)),
                  fixed((64, FLAT_F)), fixed((64, 1)),
                  fixed((SRC, 64)), fixed((SRC, 1)),
                  fixed((EH_F, 1)), fixed((EH_F, 1)),
                  fixed((64, EH_F)), fixed((64, 1)),
                  fixed((1, 64)), fixed((1, 1)),
                  fixed((1, 1)), fixed((1, 1))],
        out_specs=fixed((1, N_G)),
        out_shape=jax.ShapeDtypeStruct((1, N_G), jnp.float32),
        scratch_shapes=[pltpu.VMEM((FLAT_F, N_G), jnp.float32),
                        pltpu.VMEM((SRC, N_G), jnp.float32)],
    )(srcn_t3, batch3, shellsums,
      _col(p['se_ln_g']), _col(p['se_ln_b']),
      p['se_W1'].T, _col(p['se_b1']), p['se_W2'].T, _col(p['se_b2']),
      _col(p['eh_ln_g']), _col(p['eh_ln_b']),
      p['eh_W1'].T, _col(p['eh_b1']), p['eh_W2'].T, _scalar(p['eh_b2']),
      _scalar(p['far_gate']), _scalar(p['energy_scale']))

    return out.reshape(N_G)


# speedup vs baseline: 24.7650x; 1.1399x over previous
import jax
import jax.numpy as jnp
import numpy as np
from jax.experimental import pallas as pl
from jax.experimental.pallas import tpu as pltpu

N_G = 256
SRC = 16
NSHELL = 5
FLAT_F = NSHELL * (SRC + 3)
EH_F = 4 * SRC + FLAT_F
_CUT = 5.0
_MAX = 40.0
_RBF_STEP = float(np.linspace(_CUT, _MAX, 8)[1] - np.linspace(_CUT, _MAX, 8)[0])
_RBF_G = float(1.0 / max(_RBF_STEP ** 2, 1e-6))


def _silu_bf(x):
    xb = x.astype(jnp.bfloat16)
    return xb * jax.nn.sigmoid(xb)


def _ln_cols(x, g, b, eps=1e-5):
    m = x.mean(0, keepdims=True)
    v = ((x - m) ** 2).mean(0, keepdims=True)
    return (x - m) / jnp.sqrt(v + eps) * g + b


def _oh_bmajor(batch_ref, n):
    bids = batch_ref[0, 0, :]
    iota = jax.lax.broadcasted_iota(jnp.int32, (n, N_G), 1)
    return (bids[:, None] == iota).astype(jnp.bfloat16)


def _oh_gmajor(batch_ref, n):
    bids = batch_ref[0]
    iota = jax.lax.broadcasted_iota(jnp.int32, (N_G, n), 0)
    return (iota == bids).astype(jnp.bfloat16)


def _mm(a, b):
    return jnp.dot(a, b, preferred_element_type=jnp.float32)


def _mm_nt(a, b):
    return jax.lax.dot_general(
        a, b, (((1,), (1,)), ((), ())), preferred_element_type=jnp.float32)


def _bf(x):
    return x.astype(jnp.bfloat16)


def _k1_body(x_ref, post_ref, batch_ref, g_ref, b_ref, w1_ref, b1_ref,
             w2t_ref, b2_ref, srct_out_ref, sums_ref):
    i = pl.program_id(0)
    xb = x_ref[...]
    m = xb.mean(-1, keepdims=True)
    v = (xb * xb).mean(-1, keepdims=True) - m * m
    s = jax.lax.rsqrt(v + 1e-5)
    w1 = w1_ref[...]
    w1g = w1 * g_ref[...]
    xw = _mm(_bf(xb), _bf(w1g))
    gw = jnp.sum(w1g, axis=0, keepdims=True)
    bw = _mm(b_ref[...], w1)
    pre = s * xw - (s * m) * gw + (bw + b1_ref[...])
    a = _silu_bf(pre)
    src_t = _mm_nt(_bf(w2t_ref[...]), a) + b2_ref[...]
    src_b = _bf(src_t)
    srct_out_ref[...] = src_b[None]
    oh = _oh_bmajor(batch_ref, xb.shape[0])
    ones = jnp.ones((1, xb.shape[0]), jnp.bfloat16)
    vals = jnp.concatenate([src_b, _bf(post_ref[0]), ones], axis=0)

    @pl.when(i == 0)
    def _():
        sums_ref[...] = jnp.zeros_like(sums_ref)

    sums_ref[...] += _mm(vals, oh)


def _k2_body(srct_ref, post_ref, batch_ref, sums0_ref, sg_ref, sb_ref,
             kscr_ref, kw1t_ref, kb1_ref, kw2t_ref, kb2_ref,
             shell_ref, srcn_out_ref, gmean_s, center_s):
    i = pl.program_id(0)

    @pl.when(i == 0)
    def _():
        s0 = sums0_ref[...]
        cc = jnp.clip(s0[19:20, :], 1.0, None)
        gmean_s[...] = _bf(s0[0:16, :] / cc)
        center_s[...] = _bf(s0[16:19, :] / cc)
        shell_ref[...] = jnp.zeros_like(shell_ref)

    n = srct_ref.shape[2]
    oht = _oh_gmajor(batch_ref, n)
    gmean_b = _mm(gmean_s[...], oht)
    center_b = _mm(center_s[...], oht)
    srcn = _ln_cols(srct_ref[0].astype(jnp.float32) - gmean_b,
                    sg_ref[...], sb_ref[...])
    srcn_b = _bf(srcn)
    srcn_out_ref[...] = srcn_b[None]
    d = post_ref[0] - center_b
    r = jnp.sqrt(jnp.sum(d * d, axis=0, keepdims=True))
    screening = jax.nn.softplus(kscr_ref[0, 0])
    base = jnp.exp(-screening * r) / jnp.maximum(r, 1e-6)
    centers = _CUT + _RBF_STEP * jax.lax.broadcasted_iota(
        jnp.int32, (8, 1), 0).astype(jnp.float32)
    rbf = jnp.exp(-_RBF_G * (r - centers) ** 2)
    gate_in = jnp.concatenate([r / _CUT, r / _MAX, rbf], axis=0)
    g1 = _silu_bf(_mm(kw1t_ref[...], gate_in) + kb1_ref[...])
    lg = 1.0 + jnp.tanh(_mm(_bf(kw2t_ref[...]), g1)
                        + kb2_ref[...])
    w = base * lg

    k5 = jax.lax.broadcasted_iota(jnp.int32, (NSHELL, 1), 0)
    lows = _CUT * jnp.exp2(k5.astype(jnp.float32))
    highs = jnp.where(k5 == NSHELL - 1, jnp.inf, 2.0 * lows)
    m_all = ((r >= lows) & (r < highs)).astype(jnp.bfloat16)
    wm = _bf(w) * m_all
    rb = _bf(r)
    rows = [srcn_b * wm[s:s + 1, :] for s in range(NSHELL)]
    rows += [m_all, rb * m_all, (rb * rb) * m_all]
    vals = jnp.concatenate(rows, axis=0)
    shell_ref[...] += _mm_nt(vals, oht)


def _k3_body(srcn_ref, batch_ref, shell_ref, seg_ref, seb_ref,
             sw1t_ref, sb1_ref, sw2t_ref, sb2_ref,
             eg_ref, eb_ref, ew1t_ref, eb1_ref, ew2t_ref, eb2_ref,
             fg_ref, es_ref, out_ref, tab_s, adc_s, aux_s):
    i = pl.program_id(0)

    @pl.when(i == 0)
    def _():
        ss = shell_ref[...]
        rows = []
        for s in range(NSHELL):
            cnt = ss[80 + s:81 + s, :]
            ccs = jnp.clip(cnt, 1.0, None)
            rows.append(ss[16 * s:16 * s + 16, :] / ccs)
            rows.append(cnt)
            rows.append(ss[85 + s:86 + s, :] / ccs)
            rows.append(jnp.sqrt(ss[90 + s:91 + s, :] / ccs))
        flat = jnp.concatenate(rows, axis=0)
        t = _ln_cols(flat, seg_ref[...], seb_ref[...])
        a = _silu_bf(_mm(_bf(sw1t_ref[...]), _bf(t)) + sb1_ref[...])
        ctx = _mm(_bf(sw2t_ref[...]), a) + sb2_ref[...]
        w1gt = ew1t_ref[...] * eg_ref[...]
        wa, wb = w1gt[:, 0:16], w1gt[:, 16:32]
        wc, wd = w1gt[:, 32:48], w1gt[:, 48:64]
        we = w1gt[:, 64:159]
        gtab = _mm(_bf(wb - wd), _bf(ctx)) + _mm(_bf(we), _bf(flat))
        f1 = jnp.sum(flat, axis=0, keepdims=True)
        f2c = (jnp.sum(flat * flat, axis=0, keepdims=True)
               + 2.0 * jnp.sum(ctx * ctx, axis=0, keepdims=True))
        tab_s[...] = _bf(jnp.concatenate([ctx, gtab, f1, f2c], axis=0))
        adc_s[...] = _bf(jnp.concatenate([wa + wd, wc], axis=1))
        aux = jnp.concatenate(
            [jnp.sum(w1gt, axis=1, keepdims=True),
             _mm(ew1t_ref[...], eb_ref[...])], axis=1)
        aux_s[...] = aux
        out_ref[...] = jnp.zeros_like(out_ref)

    n = srcn_ref.shape[2]
    oht = _oh_gmajor(batch_ref, n)
    srcn_b = srcn_ref[0]
    gath = _mm(tab_s[...], oht)
    ctx_b = gath[0:16, :]
    gg = gath[16:80, :]
    f1g = gath[80:81, :]
    s2g = gath[81:82, :]
    ctxb_b = _bf(ctx_b)
    prod_b = srcn_b * ctxb_b
    adc = adc_s[...]
    z = _mm(adc[:, 0:16], srcn_b) + _mm(adc[:, 16:32], prod_b) + gg
    srcn_f = srcn_b.astype(jnp.float32)
    prod_f = prod_b.astype(jnp.float32)
    s1 = jnp.sum(srcn_f, axis=0, keepdims=True)
    s12 = jnp.sum(prod_f, axis=0, keepdims=True)
    q1 = jnp.sum(srcn_f * srcn_f, axis=0, keepdims=True)
    qq = jnp.sum(prod_f * prod_f, axis=0, keepdims=True)
    mean = (2.0 * s1 + s12 + f1g) / float(EH_F)
    msq = (2.0 * q1 + qq - 2.0 * s12 + s2g) / float(EH_F)
    inv = jax.lax.rsqrt(jnp.maximum(msq - mean * mean, 0.0) + 1e-5)
    aux = aux_s[...]
    pre = inv * (z - mean * aux[:, 0:1]) + (aux[:, 1:2] + eb1_ref[...])
    a = _silu_bf(pre)
    e = _mm(_bf(ew2t_ref[...]), a) + eb2_ref[...]
    e = e * (fg_ref[0, 0] * jnp.exp(es_ref[0, 0]))
    out_ref[...] += _mm_nt(_bf(e), oht)


def _row(v):
    return v.reshape(1, -1)


def _col(v):
    return v.reshape(-1, 1)


def _scalar(v):
    return jnp.asarray(v, jnp.float32).reshape(1, 1)


def kernel(x, pos, params, batch, num_graphs):
    p = params
    n = x.shape[0]
    blk = 2000
    for cand in (2000, 1000, 500, 200, 8):
        if n % cand == 0:
            blk = cand
            break
    nb = n // blk
    batch3 = batch.reshape(nb, 1, blk)
    pos_t3 = pos.reshape(nb, blk, 3).transpose(0, 2, 1)

    def fixed(shape):
        return pl.BlockSpec(shape, lambda i: tuple(0 for _ in shape))

    xspec = pl.BlockSpec((blk, 128), lambda i: (i, 0))
    node3 = lambda c: pl.BlockSpec((1, c, blk), lambda i: (i, 0, 0))

    src_t3, sums0 = pl.pallas_call(
        _k1_body,
        grid=(nb,),
        in_specs=[xspec, node3(3), node3(1),
                  fixed((128, 1)), fixed((1, 128)),
                  fixed((128, 64)), fixed((1, 64)),
                  fixed((SRC, 64)), fixed((SRC, 1))],
        out_specs=[node3(SRC), fixed((20, N_G))],
        out_shape=[jax.ShapeDtypeStruct((nb, SRC, blk), jnp.bfloat16),
                   jax.ShapeDtypeStruct((20, N_G), jnp.float32)],
    )(x, pos_t3, batch3,
      _col(p['in_ln_g']), _row(p['in_ln_b']),
      p['src_W1'], _row(p['src_b1']),
      p['src_W2'].T, _col(p['src_b2']))

    shellsums, srcn_t3 = pl.pallas_call(
        _k2_body,
        grid=(nb,),
        in_specs=[node3(SRC), node3(3), node3(1), fixed((20, N_G)),
                  fixed((SRC, 1)), fixed((SRC, 1)), fixed((1, 1)),
                  fixed((32, 10)), fixed((32, 1)),
                  fixed((1, 32)), fixed((1, 1))],
        out_specs=[fixed((FLAT_F, N_G)), node3(SRC)],
        out_shape=[jax.ShapeDtypeStruct((FLAT_F, N_G), jnp.float32),
                   jax.ShapeDtypeStruct((nb, SRC, blk), jnp.bfloat16)],
        scratch_shapes=[pltpu.VMEM((SRC, N_G), jnp.bfloat16),
                        pltpu.VMEM((3, N_G), jnp.bfloat16)],
    )(src_t3, pos_t3, batch3, sums0,
      _col(p['srcn_g']), _col(p['srcn_b']), _scalar(p['kscr']),
      p['kg_W1'].T, _col(p['kg_b1']), p['kg_W2'].T, _scalar(p['kg_b2']))

    out = pl.pallas_call(
        _k3_body,
        grid=(nb,),
        in_specs=[node3(SRC), node3(1), fixed((FLAT_F, N_G)),
                  fixed((FLAT_F, 1)), fixed((FLAT_F, 1)),
                  fixed((64, FLAT_F)), fixed((64, 1)),
                  fixed((SRC, 64)), fixed((SRC, 1)),
                  fixed((1, EH_F)), fixed((EH_F, 1)),
                  fixed((64, EH_F)), fixed((64, 1)),
                  fixed((1, 64)), fixed((1, 1)),
                  fixed((1, 1)), fixed((1, 1))],
        out_specs=fixed((1, N_G)),
        out_shape=jax.ShapeDtypeStruct((1, N_G), jnp.float32),
        scratch_shapes=[pltpu.VMEM((82, N_G), jnp.bfloat16),
                        pltpu.VMEM((64, 32), jnp.bfloat16),
                        pltpu.VMEM((64, 2), jnp.float32)],
    )(srcn_t3, batch3, shellsums,
      _col(p['se_ln_g']), _col(p['se_ln_b']),
      p['se_W1'].T, _col(p['se_b1']), p['se_W2'].T, _col(p['se_b2']),
      _row(p['eh_ln_g']), _col(p['eh_ln_b']),
      p['eh_W1'].T, _col(p['eh_b1']), p['eh_W2'].T, _scalar(p['eh_b2']),
      _scalar(p['far_gate']), _scalar(p['energy_scale']))

    return out.reshape(N_G)


# speedup vs baseline: 31.1127x; 1.2563x over previous
import jax
import jax.numpy as jnp
import numpy as np
from jax.experimental import pallas as pl
from jax.experimental.pallas import tpu as pltpu

N_G = 256
SRC = 16
NSHELL = 5
FLAT_F = NSHELL * (SRC + 3)
EH_F = 4 * SRC + FLAT_F
_CUT = 5.0
_MAX = 40.0
_RBF_STEP = float(np.linspace(_CUT, _MAX, 8)[1] - np.linspace(_CUT, _MAX, 8)[0])
_RBF_G = float(1.0 / max(_RBF_STEP ** 2, 1e-6))


def _silu_bf(x):
    xb = x.astype(jnp.bfloat16)
    return xb * jax.nn.sigmoid(xb)


def _ln_cols(x, g, b, eps=1e-5):
    m = x.mean(0, keepdims=True)
    v = ((x - m) ** 2).mean(0, keepdims=True)
    return (x - m) / jnp.sqrt(v + eps) * g + b


def _oh_bmajor(batch_ref, n):
    bids = batch_ref[0, 0, :]
    iota = jax.lax.broadcasted_iota(jnp.int32, (n, N_G), 1)
    return (bids[:, None] == iota).astype(jnp.bfloat16)


def _oh_gmajor(batch_ref, n):
    bids = batch_ref[0]
    iota = jax.lax.broadcasted_iota(jnp.int32, (N_G, n), 0)
    return (iota == bids).astype(jnp.bfloat16)


def _mm(a, b):
    return jnp.dot(a, b, preferred_element_type=jnp.float32)


def _mm_nt(a, b):
    return jax.lax.dot_general(
        a, b, (((1,), (1,)), ((), ())), preferred_element_type=jnp.float32)


def _bf(x):
    return x.astype(jnp.bfloat16)


def _k1_body(x_ref, post_ref, batch_ref, g_ref, b_ref, w1t_ref, b1_ref,
             w2t_ref, b2_ref, srct_out_ref, sums_ref):
    i = pl.program_id(0)
    xbb = _bf(x_ref[...])
    n = xbb.shape[0]
    ones_row = jnp.ones((1, 128), jnp.bfloat16)
    m = _mm_nt(ones_row, xbb) * (1.0 / 128.0)
    msq = _mm_nt(ones_row, xbb * xbb) * (1.0 / 128.0)
    s = jax.lax.rsqrt(jnp.maximum(msq - m * m, 0.0) + 1e-5)
    w1t = w1t_ref[...]
    w1gt = w1t * g_ref[...]
    xw = _mm_nt(_bf(w1gt), xbb)
    gw = jnp.sum(w1gt, axis=1, keepdims=True)
    bw = _mm(w1t, b_ref[...]) + b1_ref[...]
    pre = s * (xw - m * gw) + bw
    a = _silu_bf(pre)
    src_t = _mm(_bf(w2t_ref[...]), a)
    src_b = _bf(src_t + b2_ref[...])
    srct_out_ref[...] = src_b[None]
    oht = _oh_gmajor(batch_ref, n)
    ones = jnp.ones((1, n), jnp.bfloat16)
    vals = jnp.concatenate([src_b, _bf(post_ref[0]), ones], axis=0)

    @pl.when(i == 0)
    def _():
        sums_ref[...] = jnp.zeros_like(sums_ref)

    sums_ref[...] += _mm_nt(vals, oht)


def _k2_body(srct_ref, post_ref, batch_ref, sums0_ref, sg_ref, sb_ref,
             kscr_ref, kw1t_ref, kb1_ref, kw2t_ref, kb2_ref,
             shell_ref, srcn_out_ref, gmean_s, center_s):
    i = pl.program_id(0)

    @pl.when(i == 0)
    def _():
        s0 = sums0_ref[...]
        cc = jnp.clip(s0[19:20, :], 1.0, None)
        gmean_s[...] = _bf(s0[0:16, :] / cc)
        center_s[...] = _bf(s0[16:19, :] / cc)
        shell_ref[...] = jnp.zeros_like(shell_ref)

    n = srct_ref.shape[2]
    oht = _oh_gmajor(batch_ref, n)
    gmean_b = _mm(gmean_s[...], oht)
    center_b = _mm(center_s[...], oht)
    srcn = _ln_cols(srct_ref[0].astype(jnp.float32) - gmean_b,
                    sg_ref[...], sb_ref[...])
    srcn_b = _bf(srcn)
    srcn_out_ref[...] = srcn_b[None]
    d = post_ref[0] - center_b
    r = jnp.sqrt(jnp.sum(d * d, axis=0, keepdims=True))
    screening = jax.nn.softplus(kscr_ref[0, 0])
    base = jnp.exp(-screening * r) / jnp.maximum(r, 1e-6)
    centers = _CUT + _RBF_STEP * jax.lax.broadcasted_iota(
        jnp.int32, (8, 1), 0).astype(jnp.float32)
    rbf = jnp.exp(-_RBF_G * (r - centers) ** 2)
    gate_in = jnp.concatenate([r / _CUT, r / _MAX, rbf], axis=0)
    g1 = _silu_bf(_mm(_bf(kw1t_ref[...]), _bf(gate_in))
                  + kb1_ref[...])
    lg = 1.0 + jnp.tanh(_mm(_bf(kw2t_ref[...]), g1)
                        + kb2_ref[...])
    w = base * lg

    k5 = jax.lax.broadcasted_iota(jnp.int32, (NSHELL, 1), 0)
    lows = _CUT * jnp.exp2(k5.astype(jnp.float32))
    highs = jnp.where(k5 == NSHELL - 1, jnp.inf, 2.0 * lows)
    m_all = ((r >= lows) & (r < highs)).astype(jnp.bfloat16)
    wm = _bf(w) * m_all
    rb = _bf(r)
    rows = [srcn_b * wm[s:s + 1, :] for s in range(NSHELL)]
    rows += [m_all, rb * m_all, (rb * rb) * m_all]
    vals = jnp.concatenate(rows, axis=0)
    shell_ref[...] += _mm_nt(vals, oht)


def _k3_body(srcn_ref, batch_ref, shell_ref, seg_ref, seb_ref,
             sw1t_ref, sb1_ref, sw2t_ref, sb2_ref,
             eg_ref, eb_ref, ew1t_ref, eb1_ref, ew2t_ref, eb2_ref,
             fg_ref, es_ref, out_ref, tab_s, adc_s, aux_s):
    i = pl.program_id(0)

    @pl.when(i == 0)
    def _():
        ss = shell_ref[...]
        rows = []
        for s in range(NSHELL):
            cnt = ss[80 + s:81 + s, :]
            ccs = jnp.clip(cnt, 1.0, None)
            rows.append(ss[16 * s:16 * s + 16, :] / ccs)
            rows.append(cnt)
            rows.append(ss[85 + s:86 + s, :] / ccs)
            rows.append(jnp.sqrt(ss[90 + s:91 + s, :] / ccs))
        flat = jnp.concatenate(rows, axis=0)
        t = _ln_cols(flat, seg_ref[...], seb_ref[...])
        a = _silu_bf(_mm(_bf(sw1t_ref[...]), _bf(t)) + sb1_ref[...])
        ctx = _mm(_bf(sw2t_ref[...]), a) + sb2_ref[...]
        w1gt = ew1t_ref[...] * eg_ref[...]
        wa, wb = w1gt[:, 0:16], w1gt[:, 16:32]
        wc, wd = w1gt[:, 32:48], w1gt[:, 48:64]
        we = w1gt[:, 64:159]
        gtab = _mm(_bf(wb - wd), _bf(ctx)) + _mm(_bf(we), _bf(flat))
        f1 = jnp.sum(flat, axis=0, keepdims=True)
        f2c = (jnp.sum(flat * flat, axis=0, keepdims=True)
               + 2.0 * jnp.sum(ctx * ctx, axis=0, keepdims=True))
        tab_s[...] = _bf(jnp.concatenate([ctx, gtab, f1, f2c], axis=0))
        adc_s[...] = _bf(jnp.concatenate([wa + wd, wc], axis=1))
        aux = jnp.concatenate(
            [jnp.sum(w1gt, axis=1, keepdims=True),
             _mm(ew1t_ref[...], eb_ref[...])], axis=1)
        aux_s[...] = aux
        out_ref[...] = jnp.zeros_like(out_ref)

    n = srcn_ref.shape[2]
    oht = _oh_gmajor(batch_ref, n)
    srcn_b = srcn_ref[0]
    gath = _mm(tab_s[...], oht)
    ctx_b = gath[0:16, :]
    gg = gath[16:80, :]
    f1g = gath[80:81, :]
    s2g = gath[81:82, :]
    ctxb_b = _bf(ctx_b)
    prod_b = srcn_b * ctxb_b
    adc = adc_s[...]
    z = _mm(adc[:, 0:16], srcn_b) + _mm(adc[:, 16:32], prod_b) + gg
    srcn_f = srcn_b.astype(jnp.float32)
    prod_f = prod_b.astype(jnp.float32)
    s1 = jnp.sum(srcn_f, axis=0, keepdims=True)
    s12 = jnp.sum(prod_f, axis=0, keepdims=True)
    q1 = jnp.sum(srcn_f * srcn_f, axis=0, keepdims=True)
    qq = jnp.sum(prod_f * prod_f, axis=0, keepdims=True)
    mean = (2.0 * s1 + s12 + f1g) / float(EH_F)
    msq = (2.0 * q1 + qq - 2.0 * s12 + s2g) / float(EH_F)
    inv = jax.lax.rsqrt(jnp.maximum(msq - mean * mean, 0.0) + 1e-5)
    aux = aux_s[...]
    pre = inv * (z - mean * aux[:, 0:1]) + (aux[:, 1:2] + eb1_ref[...])
    a = _silu_bf(pre)
    e = _mm(_bf(ew2t_ref[...]), a) + eb2_ref[...]
    e = e * (fg_ref[0, 0] * jnp.exp(es_ref[0, 0]))
    out_ref[...] += _mm_nt(_bf(e), oht)


def _row(v):
    return v.reshape(1, -1)


def _col(v):
    return v.reshape(-1, 1)


def _scalar(v):
    return jnp.asarray(v, jnp.float32).reshape(1, 1)


def kernel(x, pos, params, batch, num_graphs):
    p = params
    n = x.shape[0]
    blk = 2000
    for cand in (4000, 2000, 1000, 500, 200, 8):
        if n % cand == 0:
            blk = cand
            break
    nb = n // blk
    batch3 = batch.reshape(nb, 1, blk)
    pos_t3 = pos.reshape(nb, blk, 3).transpose(0, 2, 1)

    def fixed(shape):
        return pl.BlockSpec(shape, lambda i: tuple(0 for _ in shape))

    xspec = pl.BlockSpec((blk, 128), lambda i: (i, 0))
    node3 = lambda c: pl.BlockSpec((1, c, blk), lambda i: (i, 0, 0))

    src_t3, sums0 = pl.pallas_call(
        _k1_body,
        grid=(nb,),
        in_specs=[xspec, node3(3), node3(1),
                  fixed((1, 128)), fixed((128, 1)),
                  fixed((64, 128)), fixed((64, 1)),
                  fixed((SRC, 64)), fixed((SRC, 1))],
        out_specs=[node3(SRC), fixed((20, N_G))],
        out_shape=[jax.ShapeDtypeStruct((nb, SRC, blk), jnp.bfloat16),
                   jax.ShapeDtypeStruct((20, N_G), jnp.float32)],
    )(x, pos_t3, batch3,
      _row(p['in_ln_g']), _col(p['in_ln_b']),
      p['src_W1'].T, _col(p['src_b1']),
      p['src_W2'].T, _col(p['src_b2']))

    shellsums, srcn_t3 = pl.pallas_call(
        _k2_body,
        grid=(nb,),
        in_specs=[node3(SRC), node3(3), node3(1), fixed((20, N_G)),
                  fixed((SRC, 1)), fixed((SRC, 1)), fixed((1, 1)),
                  fixed((32, 10)), fixed((32, 1)),
                  fixed((1, 32)), fixed((1, 1))],
        out_specs=[fixed((FLAT_F, N_G)), node3(SRC)],
        out_shape=[jax.ShapeDtypeStruct((FLAT_F, N_G), jnp.float32),
                   jax.ShapeDtypeStruct((nb, SRC, blk), jnp.bfloat16)],
        scratch_shapes=[pltpu.VMEM((SRC, N_G), jnp.bfloat16),
                        pltpu.VMEM((3, N_G), jnp.bfloat16)],
    )(src_t3, pos_t3, batch3, sums0,
      _col(p['srcn_g']), _col(p['srcn_b']), _scalar(p['kscr']),
      p['kg_W1'].T, _col(p['kg_b1']), p['kg_W2'].T, _scalar(p['kg_b2']))

    out = pl.pallas_call(
        _k3_body,
        grid=(nb,),
        in_specs=[node3(SRC), node3(1), fixed((FLAT_F, N_G)),
                  fixed((FLAT_F, 1)), fixed((FLAT_F, 1)),
                  fixed((64, FLAT_F)), fixed((64, 1)),
                  fixed((SRC, 64)), fixed((SRC, 1)),
                  fixed((1, EH_F)), fixed((EH_F, 1)),
                  fixed((64, EH_F)), fixed((64, 1)),
                  fixed((1, 64)), fixed((1, 1)),
                  fixed((1, 1)), fixed((1, 1))],
        out_specs=fixed((1, N_G)),
        out_shape=jax.ShapeDtypeStruct((1, N_G), jnp.float32),
        scratch_shapes=[pltpu.VMEM((82, N_G), jnp.bfloat16),
                        pltpu.VMEM((64, 32), jnp.bfloat16),
                        pltpu.VMEM((64, 2), jnp.float32)],
    )(srcn_t3, batch3, shellsums,
      _col(p['se_ln_g']), _col(p['se_ln_b']),
      p['se_W1'].T, _col(p['se_b1']), p['se_W2'].T, _col(p['se_b2']),
      _row(p['eh_ln_g']), _col(p['eh_ln_b']),
      p['eh_W1'].T, _col(p['eh_b1']), p['eh_W2'].T, _scalar(p['eh_b2']),
      _scalar(p['far_gate']), _scalar(p['energy_scale']))

    return out.reshape(N_G)


# speedup vs baseline: 31.1988x; 1.0028x over previous
import jax
import jax.numpy as jnp
import numpy as np
from jax.experimental import pallas as pl
from jax.experimental.pallas import tpu as pltpu

N_G = 256
SRC = 16
NSHELL = 5
FLAT_F = NSHELL * (SRC + 3)
EH_F = 4 * SRC + FLAT_F
_CUT = 5.0
_MAX = 40.0
_RBF_STEP = float(np.linspace(_CUT, _MAX, 8)[1] - np.linspace(_CUT, _MAX, 8)[0])
_RBF_G = float(1.0 / max(_RBF_STEP ** 2, 1e-6))


def _silu_bf(x):
    xb = x.astype(jnp.bfloat16)
    return xb * jax.nn.sigmoid(xb)


def _ln_cols(x, g, b, eps=1e-5):
    m = x.mean(0, keepdims=True)
    v = ((x - m) ** 2).mean(0, keepdims=True)
    return (x - m) / jnp.sqrt(v + eps) * g + b


def _oh_bmajor(batch_ref, n):
    bids = batch_ref[0, 0, :]
    iota = jax.lax.broadcasted_iota(jnp.int32, (n, N_G), 1)
    return (bids[:, None] == iota).astype(jnp.bfloat16)


def _oh_gmajor(batch_ref, n):
    bids = batch_ref[0]
    iota = jax.lax.broadcasted_iota(jnp.int32, (N_G, n), 0)
    return (iota == bids).astype(jnp.bfloat16)


def _mm(a, b):
    return jnp.dot(a, b, preferred_element_type=jnp.float32)


def _mm_nt(a, b):
    return jax.lax.dot_general(
        a, b, (((1,), (1,)), ((), ())), preferred_element_type=jnp.float32)


def _bf(x):
    return x.astype(jnp.bfloat16)


def _hilo(x):
    hi = _bf(x)
    lo = _bf(x - hi.astype(jnp.float32))
    return jnp.concatenate([hi, lo], axis=0)


def _k1_body(x_ref, post_ref, batch_ref, g_ref, b_ref, w1t_ref, b1_ref,
             w2t_ref, b2_ref, srct_out_ref, sums_ref):
    i = pl.program_id(0)
    xbb = _bf(x_ref[...])
    n = xbb.shape[0]
    ones_row = jnp.ones((1, 128), jnp.bfloat16)
    m = _mm_nt(ones_row, xbb) * (1.0 / 128.0)
    msq = _mm_nt(ones_row, xbb * xbb) * (1.0 / 128.0)
    s = jax.lax.rsqrt(jnp.maximum(msq - m * m, 0.0) + 1e-5)
    w1t = w1t_ref[...]
    w1gt = w1t * g_ref[...]
    xw = _mm_nt(_bf(w1gt), xbb)
    gw = jnp.sum(w1gt, axis=1, keepdims=True)
    bw = _mm(w1t, b_ref[...]) + b1_ref[...]
    pre = s * (xw - m * gw) + bw
    a = _silu_bf(pre)
    src_t = _mm(_bf(w2t_ref[...]), a)
    src_f = src_t + b2_ref[...]
    srct_out_ref[...] = src_f[None]
    src_b = _bf(src_f)
    oht = _oh_gmajor(batch_ref, n)
    ones = jnp.ones((1, n), jnp.bfloat16)
    pos_f = post_ref[0]
    pos_hi = _bf(pos_f)
    pos_lo = _bf(pos_f - pos_hi.astype(jnp.float32))
    vals = jnp.concatenate([src_b, pos_hi, pos_lo, ones], axis=0)

    @pl.when(i == 0)
    def _():
        sums_ref[...] = jnp.zeros_like(sums_ref)

    sums_ref[...] += _mm_nt(vals, oht)


def _k2_body(srct_ref, post_ref, batch_ref, sums0_ref, sg_ref, sb_ref,
             kscr_ref, kw1t_ref, kb1_ref, kw2t_ref, kb2_ref,
             shell_ref, srcn_out_ref, cat_s):
    i = pl.program_id(0)

    @pl.when(i == 0)
    def _():
        s0 = sums0_ref[...]
        cc = jnp.clip(s0[22:23, :], 1.0, None)
        center = (s0[16:19, :] + s0[19:22, :]) / cc
        cat = jnp.concatenate([s0[0:16, :] / cc, center], axis=0)
        cat_s[...] = _hilo(cat)
        shell_ref[...] = jnp.zeros_like(shell_ref)

    n = srct_ref.shape[2]
    oht = _oh_gmajor(batch_ref, n)
    g2 = _mm(cat_s[...], oht)
    gath = g2[0:19, :] + g2[19:38, :]
    gmean_b = gath[0:16, :]
    center_b = gath[16:19, :]
    srcn = _ln_cols(srct_ref[0] - gmean_b, sg_ref[...], sb_ref[...])
    srcn_b = _bf(srcn)
    srcn_out_ref[...] = srcn[None]
    d = post_ref[0] - center_b
    r = jnp.sqrt(jnp.sum(d * d, axis=0, keepdims=True))
    screening = jax.nn.softplus(kscr_ref[0, 0])
    base = jnp.exp(-screening * r) / jnp.maximum(r, 1e-6)
    centers = _CUT + _RBF_STEP * jax.lax.broadcasted_iota(
        jnp.int32, (8, 1), 0).astype(jnp.float32)
    rbf = jnp.exp(-_RBF_G * (r - centers) ** 2)
    gate_in = jnp.concatenate([r / _CUT, r / _MAX, rbf], axis=0)
    g1 = _silu_bf(_mm(_bf(kw1t_ref[...]), _bf(gate_in))
                  + kb1_ref[...])
    lg = 1.0 + jnp.tanh(_mm(_bf(kw2t_ref[...]), g1)
                        + kb2_ref[...])
    w = base * lg

    k5 = jax.lax.broadcasted_iota(jnp.int32, (NSHELL, 1), 0)
    lows = _CUT * jnp.exp2(k5.astype(jnp.float32))
    highs = jnp.where(k5 == NSHELL - 1, jnp.inf, 2.0 * lows)
    m_all = ((r >= lows) & (r < highs)).astype(jnp.bfloat16)
    wm = _bf(w) * m_all
    rb = _bf(r)
    rows = [srcn_b * wm[s:s + 1, :] for s in range(NSHELL)]
    rows += [m_all, rb * m_all, (rb * rb) * m_all]
    vals = jnp.concatenate(rows, axis=0)
    shell_ref[...] += _mm_nt(vals, oht)


def _k3_body(srcn_ref, batch_ref, shell_ref, seg_ref, seb_ref,
             sw1t_ref, sb1_ref, sw2t_ref, sb2_ref,
             eg_ref, eb_ref, ew1t_ref, eb1_ref, ew2t_ref, eb2_ref,
             fg_ref, es_ref, out_ref, tab_s, adc_s, aux_s):
    i = pl.program_id(0)

    @pl.when(i == 0)
    def _():
        ss = shell_ref[...]
        rows = []
        for s in range(NSHELL):
            cnt = ss[80 + s:81 + s, :]
            ccs = jnp.clip(cnt, 1.0, None)
            rows.append(ss[16 * s:16 * s + 16, :] / ccs)
            rows.append(cnt)
            rows.append(ss[85 + s:86 + s, :] / ccs)
            rows.append(jnp.sqrt(ss[90 + s:91 + s, :] / ccs))
        flat = jnp.concatenate(rows, axis=0)
        t = _ln_cols(flat, seg_ref[...], seb_ref[...])
        a = _silu_bf(_mm(_bf(sw1t_ref[...]), _bf(t)) + sb1_ref[...])
        ctx = _mm(_bf(sw2t_ref[...]), a) + sb2_ref[...]
        w1gt = ew1t_ref[...] * eg_ref[...]
        wa, wb = w1gt[:, 0:16], w1gt[:, 16:32]
        wc, wd = w1gt[:, 32:48], w1gt[:, 48:64]
        we = w1gt[:, 64:159]
        gtab = _mm(_bf(wb - wd), _bf(ctx)) + _mm(_bf(we), _bf(flat))
        f1 = jnp.sum(flat, axis=0, keepdims=True)
        f2c = (jnp.sum(flat * flat, axis=0, keepdims=True)
               + 2.0 * jnp.sum(ctx * ctx, axis=0, keepdims=True))
        tab_s[...] = _hilo(
            jnp.concatenate([ctx, gtab, f1, f2c], axis=0))
        adc_s[...] = _bf(jnp.concatenate([wa + wd, wc], axis=1))
        aux = jnp.concatenate(
            [jnp.sum(w1gt, axis=1, keepdims=True),
             _mm(ew1t_ref[...], eb_ref[...])], axis=1)
        aux_s[...] = aux
        out_ref[...] = jnp.zeros_like(out_ref)

    n = srcn_ref.shape[2]
    oht = _oh_gmajor(batch_ref, n)
    srcn_f = srcn_ref[0]
    srcn_b = _bf(srcn_f)
    g2 = _mm(tab_s[...], oht)
    gath = g2[0:82, :] + g2[82:164, :]
    ctx_b = gath[0:16, :]
    gg = gath[16:80, :]
    f1g = gath[80:81, :]
    s2g = gath[81:82, :]
    ctxb_b = _bf(ctx_b)
    prod_b = srcn_b * ctxb_b
    adc = adc_s[...]
    z = _mm(adc[:, 0:16], srcn_b) + _mm(adc[:, 16:32], prod_b) + gg
    prod_f = srcn_f * ctx_b
    s1 = jnp.sum(srcn_f, axis=0, keepdims=True)
    s12 = jnp.sum(prod_f, axis=0, keepdims=True)
    q1 = jnp.sum(srcn_f * srcn_f, axis=0, keepdims=True)
    qq = jnp.sum(prod_f * prod_f, axis=0, keepdims=True)
    mean = (2.0 * s1 + s12 + f1g) / float(EH_F)
    msq = (2.0 * q1 + qq - 2.0 * s12 + s2g) / float(EH_F)
    inv = jax.lax.rsqrt(jnp.maximum(msq - mean * mean, 0.0) + 1e-5)
    aux = aux_s[...]
    pre = inv * (z - mean * aux[:, 0:1]) + (aux[:, 1:2] + eb1_ref[...])
    a = _silu_bf(pre)
    e = _mm(_bf(ew2t_ref[...]), a) + eb2_ref[...]
    e = e * (fg_ref[0, 0] * jnp.exp(es_ref[0, 0]))
    eo = _mm_nt(_hilo(e), oht)
    out_ref[...] += eo[0:1, :] + eo[1:2, :]


def _row(v):
    return v.reshape(1, -1)


def _col(v):
    return v.reshape(-1, 1)


def _scalar(v):
    return jnp.asarray(v, jnp.float32).reshape(1, 1)


def kernel(x, pos, params, batch, num_graphs):
    p = params
    n = x.shape[0]
    blk = 2000
    for cand in (5000, 4000, 2000, 1000, 500, 200, 8):
        if n % cand == 0:
            blk = cand
            break
    nb = n // blk
    batch3 = batch.reshape(nb, 1, blk)
    pos_t3 = pos.reshape(nb, blk, 3).transpose(0, 2, 1)

    def fixed(shape):
        return pl.BlockSpec(shape, lambda i: tuple(0 for _ in shape))

    xspec = pl.BlockSpec((blk, 128), lambda i: (i, 0))
    node3 = lambda c: pl.BlockSpec((1, c, blk), lambda i: (i, 0, 0))

    src_t3, sums0 = pl.pallas_call(
        _k1_body,
        grid=(nb,),
        in_specs=[xspec, node3(3), node3(1),
                  fixed((1, 128)), fixed((128, 1)),
                  fixed((64, 128)), fixed((64, 1)),
                  fixed((SRC, 64)), fixed((SRC, 1))],
        out_specs=[node3(SRC), fixed((23, N_G))],
        out_shape=[jax.ShapeDtypeStruct((nb, SRC, blk), jnp.float32),
                   jax.ShapeDtypeStruct((23, N_G), jnp.float32)],
    )(x, pos_t3, batch3,
      _row(p['in_ln_g']), _col(p['in_ln_b']),
      p['src_W1'].T, _col(p['src_b1']),
      p['src_W2'].T, _col(p['src_b2']))

    shellsums, srcn_t3 = pl.pallas_call(
        _k2_body,
        grid=(nb,),
        in_specs=[node3(SRC), node3(3), node3(1), fixed((23, N_G)),
                  fixed((SRC, 1)), fixed((SRC, 1)), fixed((1, 1)),
                  fixed((32, 10)), fixed((32, 1)),
                  fixed((1, 32)), fixed((1, 1))],
        out_specs=[fixed((FLAT_F, N_G)), node3(SRC)],
        out_shape=[jax.ShapeDtypeStruct((FLAT_F, N_G), jnp.float32),
                   jax.ShapeDtypeStruct((nb, SRC, blk), jnp.float32)],
        scratch_shapes=[pltpu.VMEM((38, N_G), jnp.bfloat16)],
    )(src_t3, pos_t3, batch3, sums0,
      _col(p['srcn_g']), _col(p['srcn_b']), _scalar(p['kscr']),
      p['kg_W1'].T, _col(p['kg_b1']), p['kg_W2'].T, _scalar(p['kg_b2']))

    out = pl.pallas_call(
        _k3_body,
        grid=(nb,),
        in_specs=[node3(SRC), node3(1), fixed((FLAT_F, N_G)),
                  fixed((FLAT_F, 1)), fixed((FLAT_F, 1)),
                  fixed((64, FLAT_F)), fixed((64, 1)),
                  fixed((SRC, 64)), fixed((SRC, 1)),
                  fixed((1, EH_F)), fixed((EH_F, 1)),
                  fixed((64, EH_F)), fixed((64, 1)),
                  fixed((1, 64)), fixed((1, 1)),
                  fixed((1, 1)), fixed((1, 1))],
        out_specs=fixed((1, N_G)),
        out_shape=jax.ShapeDtypeStruct((1, N_G), jnp.float32),
        scratch_shapes=[pltpu.VMEM((164, N_G), jnp.bfloat16),
                        pltpu.VMEM((64, 32), jnp.bfloat16),
                        pltpu.VMEM((64, 2), jnp.float32)],
    )(srcn_t3, batch3, shellsums,
      _col(p['se_ln_g']), _col(p['se_ln_b']),
      p['se_W1'].T, _col(p['se_b1']), p['se_W2'].T, _col(p['se_b2']),
      _row(p['eh_ln_g']), _col(p['eh_ln_b']),
      p['eh_W1'].T, _col(p['eh_b1']), p['eh_W2'].T, _scalar(p['eh_b2']),
      _scalar(p['far_gate']), _scalar(p['energy_scale']))

    return out.reshape(N_G)
